# Initial kernel scaffold; baseline (speedup 1.0000x reference)
import jax
import jax.numpy as jnp
from jax.experimental import pallas as pl


def kernel(x, edge_index, W1, b1, W2, b2, Wc, bc):
    raise NotImplementedError("write your pallas kernel here")


# speedup vs baseline: 26.5276x; 26.5276x over previous
import functools

import jax
import jax.numpy as jnp
from jax import lax
from jax.experimental import pallas as pl
from jax.experimental.pallas import tpu as pltpu
from jax.experimental.pallas import tpu_sc as plsc

N = 10000
NP = 10240
E = 320000
EPB = 128
BPT = 80
NT = 32
EPT = BPT * EPB
EP = NT * EPT
SPT = NP // 16
D = 128


def _sc_mesh():
    return plsc.VectorSubcoreMesh(core_axis_name="c", subcore_axis_name="s")


@functools.partial(
    pl.kernel,
    out_type=jax.ShapeDtypeStruct((NT, SPT), jnp.float32),
    mesh=_sc_mesh(),
    scratch_types=[
        pltpu.VMEM((BPT, EPB), jnp.int32),
        pltpu.VMEM((EPB,), jnp.float32),
        pltpu.VMEM_SHARED((NP,), jnp.float32),
    ],
)
def _deg_kernel(dst_hbm, ones_hbm, zeros_hbm, out_hbm, idx_d, ones_v, acc):
    c = lax.axis_index("c")
    s = lax.axis_index("s")
    wid = c * 16 + s
    rsl = pl.ds(s * SPT, SPT)
    pltpu.sync_copy(zeros_hbm.at[rsl], acc.at[rsl])
    pltpu.sync_copy(ones_hbm, ones_v)
    pltpu.sync_copy(dst_hbm.at[pl.ds(wid * BPT, BPT)], idx_d)
    plsc.subcore_barrier()

    def body(j, carry):
        pltpu.sync_copy(ones_v, acc.at[idx_d.at[j]], add=True)
        return carry

    lax.fori_loop(0, BPT, body, 0)
    plsc.subcore_barrier()
    pltpu.sync_copy(acc.at[rsl], out_hbm.at[wid])


@functools.partial(
    pl.kernel,
    out_type=jax.ShapeDtypeStruct((NT, SPT, D), jnp.float32),
    mesh=_sc_mesh(),
    scratch_types=[
        pltpu.VMEM((BPT // 2, EPB), jnp.int32),
        pltpu.VMEM((BPT // 2, EPB), jnp.int32),
        [pltpu.VMEM((EPB, D), jnp.float32)] * 2,
        pltpu.VMEM_SHARED((NP, D), jnp.float32),
        pltpu.SemaphoreType.DMA,
        pltpu.SemaphoreType.DMA,
    ],
)
def _agg_kernel(src_hbm, dst_hbm, y_hbm, zeros_hbm, out_hbm,
                idx_s, idx_d, rows, acc, sem, semi):
    c = lax.axis_index("c")
    s = lax.axis_index("s")
    wid = c * 16 + s
    rsl = pl.ds(s * SPT, SPT)
    CH = BPT // 2
    pltpu.sync_copy(zeros_hbm.at[rsl], acc.at[rsl])
    plsc.subcore_barrier()

    for chunk in range(2):
        base = wid * BPT + chunk * CH
        h1 = pltpu.async_copy(src_hbm.at[pl.ds(base, CH)], idx_s, semi)
        h2 = pltpu.async_copy(dst_hbm.at[pl.ds(base, CH)], idx_d, semi)
        h1.wait()
        h2.wait()

        def body(i, carry):
            j0 = i * 2
            g0 = pltpu.async_copy(y_hbm.at[idx_s.at[j0]], rows[0], sem)
            g1 = pltpu.async_copy(y_hbm.at[idx_s.at[j0 + 1]], rows[1], sem)
            g0.wait()
            pltpu.sync_copy(rows[0], acc.at[idx_d.at[j0]], add=True)
            g1.wait()
            pltpu.sync_copy(rows[1], acc.at[idx_d.at[j0 + 1]], add=True)
            return carry

        lax.fori_loop(0, CH // 2, body, 0)

    plsc.subcore_barrier()
    pltpu.sync_copy(acc.at[rsl], out_hbm.at[wid])


RB = 2000


def _dis(degp):
    return lax.rsqrt(degp[:, 0:1] + degp[:, 1:2] + 1.0)


def _pre_body(degp_ref, x_ref, w1_ref, y1_ref):
    dis = _dis(degp_ref[...])
    xw = jnp.dot(x_ref[...], w1_ref[...], preferred_element_type=jnp.float32)
    y1_ref[...] = dis * xw


def _mid_body(degp_ref, agg_ref, y1_ref, b1_ref, w2_ref, y2_ref):
    dis = _dis(degp_ref[...])
    a = agg_ref[0] + agg_ref[1] + y1_ref[...]
    h1 = jnp.maximum(dis * a + b1_ref[...], 0.0)
    y2_ref[...] = dis * jnp.dot(h1, w2_ref[...], preferred_element_type=jnp.float32)


def _post_body(degp_ref, agg_ref, y2_ref, b2_ref, wc_ref, bc_ref, out_ref):
    dis = _dis(degp_ref[...])
    a = agg_ref[0] + agg_ref[1] + y2_ref[...]
    h2 = jnp.maximum(dis * a + b2_ref[...], 0.0)
    out_ref[...] = (
        jnp.dot(h2, wc_ref[...], preferred_element_type=jnp.float32) + bc_ref[...])


def _row_spec(cols):
    return pl.BlockSpec((RB, cols), lambda i: (i, 0))


def _full_spec(shape):
    nd = len(shape)
    return pl.BlockSpec(shape, lambda i, _nd=nd: (0,) * _nd)


def _agg_spec(cols):
    return pl.BlockSpec((2, RB, cols), lambda i: (0, i, 0))


_GRID = N // RB

_pre = pl.pallas_call(
    _pre_body,
    grid=(_GRID,),
    in_specs=[_row_spec(2), _row_spec(128), _full_spec((128, D))],
    out_specs=_row_spec(D),
    out_shape=jax.ShapeDtypeStruct((N, D), jnp.float32),
)

_mid = pl.pallas_call(
    _mid_body,
    grid=(_GRID,),
    in_specs=[_row_spec(2), _agg_spec(D), _row_spec(D),
              _full_spec((1, D)), _full_spec((D, D))],
    out_specs=_row_spec(D),
    out_shape=jax.ShapeDtypeStruct((N, D), jnp.float32),
)

_post = pl.pallas_call(
    _post_body,
    grid=(_GRID,),
    in_specs=[_row_spec(2), _agg_spec(D), _row_spec(D),
              _full_spec((1, D)), _full_spec((D, 10)), _full_spec((1, 10))],
    out_specs=_row_spec(10),
    out_shape=jax.ShapeDtypeStruct((N, 10), jnp.float32),
)


def kernel(x, edge_index, W1, b1, W2, b2, Wc, bc):
    npad = EP - E
    pad_src = jnp.arange(npad, dtype=jnp.int32) % N
    pad_dst = N + jnp.arange(npad, dtype=jnp.int32) % (NP - N)
    src = jnp.concatenate([edge_index[0].astype(jnp.int32), pad_src])
    dst = jnp.concatenate([edge_index[1].astype(jnp.int32), pad_dst])
    src = src.reshape(NT * BPT, EPB)
    dst = dst.reshape(NT * BPT, EPB)

    ones = jnp.ones((EPB,), jnp.float32)
    zeros1 = jnp.zeros((NP,), jnp.float32)
    zerosD = jnp.zeros((NP, D), jnp.float32)

    w1p = jnp.zeros((128, D), jnp.float32).at[:, :64].set(W1)
    b1p = jnp.zeros((1, D), jnp.float32).at[0, :64].set(b1)
    w2p = jnp.zeros((D, D), jnp.float32).at[:64, :32].set(W2)
    b2p = jnp.zeros((1, D), jnp.float32).at[0, :32].set(b2)
    wcp = jnp.zeros((D, 10), jnp.float32).at[:32, :].set(Wc)

    degp = _deg_kernel(dst, ones, zeros1)
    degp = degp.reshape(2, NP)[:, :N].T

    y1 = _pre(degp, x, w1p)
    agg1 = _agg_kernel(src, dst, y1, zerosD).reshape(2, NP, D)[:, :N]
    y2 = _mid(degp, agg1, y1, b1p, w2p)
    agg2 = _agg_kernel(src, dst, y2, zerosD).reshape(2, NP, D)[:, :N]
    return _post(degp, agg2, y2, b2p, wcp, bc.reshape(1, 10))


# speedup vs baseline: 28.1256x; 1.0602x over previous
import functools

import jax
import jax.numpy as jnp
from jax import lax
from jax.experimental import pallas as pl
from jax.experimental.pallas import tpu as pltpu
from jax.experimental.pallas import tpu_sc as plsc

N = 10000
NP = 10240
E = 320000
EPB = 128
BPT = 80
NT = 32
EPT = BPT * EPB
EP = NT * EPT
SPT = NP // 16
D = 128


def _sc_mesh():
    return plsc.VectorSubcoreMesh(core_axis_name="c", subcore_axis_name="s")


@functools.partial(
    pl.kernel,
    out_type=jax.ShapeDtypeStruct((NT, SPT), jnp.float32),
    mesh=_sc_mesh(),
    scratch_types=[
        pltpu.VMEM((BPT, EPB), jnp.int32),
        pltpu.VMEM((EPB,), jnp.float32),
        pltpu.VMEM_SHARED((NP,), jnp.float32),
        pltpu.SemaphoreType.DMA,
    ],
)
def _deg_kernel(dst_hbm, ones_hbm, zeros_hbm, out_hbm, idx_d, ones_v, acc, sem):
    c = lax.axis_index("c")
    s = lax.axis_index("s")
    wid = c * 16 + s
    rsl = pl.ds(s * SPT, SPT)
    pltpu.sync_copy(zeros_hbm.at[rsl], acc.at[rsl])
    pltpu.sync_copy(ones_hbm, ones_v)
    pltpu.sync_copy(dst_hbm.at[pl.ds(wid * BPT, BPT)], idx_d)
    plsc.subcore_barrier()

    def body(j, carry):
        pltpu.async_copy(ones_v, acc.at[idx_d.at[j]], sem, add=True)
        return carry

    lax.fori_loop(0, BPT, body, 0)

    def drain(j, carry):
        pltpu.make_async_copy(ones_hbm, ones_v, sem).wait()
        return carry

    lax.fori_loop(0, BPT, drain, 0)
    plsc.subcore_barrier()
    pltpu.sync_copy(acc.at[rsl], out_hbm.at[wid])


@functools.partial(
    pl.kernel,
    out_type=jax.ShapeDtypeStruct((NT, SPT, D), jnp.float32),
    mesh=_sc_mesh(),
    scratch_types=[
        pltpu.VMEM((BPT // 2, EPB), jnp.int32),
        pltpu.VMEM((BPT // 2, EPB), jnp.int32),
        [pltpu.VMEM((EPB, D), jnp.float32)] * 2,
        pltpu.VMEM_SHARED((NP, D), jnp.float32),
        pltpu.SemaphoreType.DMA,
        pltpu.SemaphoreType.DMA,
    ],
)
def _agg_kernel(src_hbm, dst_hbm, y_hbm, zeros_hbm, out_hbm,
                idx_s, idx_d, rows, acc, sem, semi):
    c = lax.axis_index("c")
    s = lax.axis_index("s")
    wid = c * 16 + s
    rsl = pl.ds(s * SPT, SPT)
    CH = BPT // 2
    pltpu.sync_copy(zeros_hbm.at[rsl], acc.at[rsl])
    plsc.subcore_barrier()

    for chunk in range(2):
        base = wid * BPT + chunk * CH
        h1 = pltpu.async_copy(src_hbm.at[pl.ds(base, CH)], idx_s, semi)
        h2 = pltpu.async_copy(dst_hbm.at[pl.ds(base, CH)], idx_d, semi)
        h1.wait()
        h2.wait()

        def drain2():
            pltpu.make_async_copy(y_hbm.at[pl.ds(0, EPB)], rows[0], semi).wait()
            pltpu.make_async_copy(y_hbm.at[pl.ds(0, EPB)], rows[1], semi).wait()

        def body(i, carry):
            j0 = i * 2

            @pl.when(i > 0)
            def _():
                drain2()

            g0 = pltpu.async_copy(y_hbm.at[idx_s.at[j0]], rows[0], sem)
            g1 = pltpu.async_copy(y_hbm.at[idx_s.at[j0 + 1]], rows[1], sem)
            g0.wait()
            pltpu.async_copy(rows[0], acc.at[idx_d.at[j0]], semi, add=True)
            g1.wait()
            pltpu.async_copy(rows[1], acc.at[idx_d.at[j0 + 1]], semi, add=True)
            return carry

        lax.fori_loop(0, CH // 2, body, 0)
        drain2()

    plsc.subcore_barrier()
    pltpu.sync_copy(acc.at[rsl], out_hbm.at[wid])


RB = 2000


def _dis(degp):
    return lax.rsqrt(degp[:, 0:1] + degp[:, 1:2] + 1.0)


def _mm_body(x_ref, w1_ref, xw_ref):
    xw_ref[...] = jnp.dot(
        x_ref[...], w1_ref[...], preferred_element_type=jnp.float32)


def _scale_body(degp_ref, xw_ref, y1_ref):
    y1_ref[...] = _dis(degp_ref[...]) * xw_ref[...]


def _mid_body(degp_ref, agg_ref, y1_ref, b1_ref, w2_ref, y2_ref):
    dis = _dis(degp_ref[...])
    a = agg_ref[0] + agg_ref[1] + y1_ref[...]
    h1 = jnp.maximum(dis * a + b1_ref[...], 0.0)
    y2_ref[...] = dis * jnp.dot(h1, w2_ref[...], preferred_element_type=jnp.float32)


def _post_body(degp_ref, agg_ref, y2_ref, b2_ref, wc_ref, bc_ref, out_ref):
    dis = _dis(degp_ref[...])
    a = agg_ref[0] + agg_ref[1] + y2_ref[...]
    h2 = jnp.maximum(dis * a + b2_ref[...], 0.0)
    out_ref[...] = (
        jnp.dot(h2, wc_ref[...], preferred_element_type=jnp.float32) + bc_ref[...])


def _row_spec(cols):
    return pl.BlockSpec((RB, cols), lambda i: (i, 0))


def _full_spec(shape):
    nd = len(shape)
    return pl.BlockSpec(shape, lambda i, _nd=nd: (0,) * _nd)


def _agg_spec(cols):
    return pl.BlockSpec((2, RB, cols), lambda i: (0, i, 0))


_GRID = N // RB

_mm = pl.pallas_call(
    _mm_body,
    grid=(_GRID,),
    in_specs=[_row_spec(128), _full_spec((128, D))],
    out_specs=_row_spec(D),
    out_shape=jax.ShapeDtypeStruct((N, D), jnp.float32),
)

_scale = pl.pallas_call(
    _scale_body,
    grid=(_GRID,),
    in_specs=[_row_spec(2), _row_spec(D)],
    out_specs=_row_spec(D),
    out_shape=jax.ShapeDtypeStruct((N, D), jnp.float32),
)

_mid = pl.pallas_call(
    _mid_body,
    grid=(_GRID,),
    in_specs=[_row_spec(2), _agg_spec(D), _row_spec(D),
              _full_spec((1, D)), _full_spec((D, D))],
    out_specs=_row_spec(D),
    out_shape=jax.ShapeDtypeStruct((N, D), jnp.float32),
)

_post = pl.pallas_call(
    _post_body,
    grid=(_GRID,),
    in_specs=[_row_spec(2), _agg_spec(D), _row_spec(D),
              _full_spec((1, D)), _full_spec((D, 10)), _full_spec((1, 10))],
    out_specs=_row_spec(10),
    out_shape=jax.ShapeDtypeStruct((N, 10), jnp.float32),
)


def kernel(x, edge_index, W1, b1, W2, b2, Wc, bc):
    npad = EP - E
    pad_src = jnp.arange(npad, dtype=jnp.int32) % N
    pad_dst = N + jnp.arange(npad, dtype=jnp.int32) % (NP - N)
    src = jnp.concatenate([edge_index[0].astype(jnp.int32), pad_src])
    dst = jnp.concatenate([edge_index[1].astype(jnp.int32), pad_dst])
    src = src.reshape(NT * BPT, EPB)
    dst = dst.reshape(NT * BPT, EPB)

    ones = jnp.ones((EPB,), jnp.float32)
    zeros1 = jnp.zeros((NP,), jnp.float32)
    zerosD = jnp.zeros((NP, D), jnp.float32)

    w1p = jnp.zeros((128, D), jnp.float32).at[:, :64].set(W1)
    b1p = jnp.zeros((1, D), jnp.float32).at[0, :64].set(b1)
    w2p = jnp.zeros((D, D), jnp.float32).at[:64, :32].set(W2)
    b2p = jnp.zeros((1, D), jnp.float32).at[0, :32].set(b2)
    wcp = jnp.zeros((D, 10), jnp.float32).at[:32, :].set(Wc)

    degp = _deg_kernel(dst, ones, zeros1)
    degp = degp.reshape(2, NP).T
    xw1 = _mm(x, w1p)
    y1 = _scale(degp, xw1)
    agg1 = _agg_kernel(src, dst, y1, zerosD).reshape(2, NP, D)
    y2 = _mid(degp, agg1, y1, b1p, w2p)
    agg2 = _agg_kernel(src, dst, y2, zerosD).reshape(2, NP, D)
    return _post(degp, agg2, y2, b2p, wcp, bc.reshape(1, 10))


# speedup vs baseline: 39.5908x; 1.4076x over previous
import functools

import jax
import jax.numpy as jnp
from jax import lax
from jax.experimental import pallas as pl
from jax.experimental.pallas import tpu as pltpu
from jax.experimental.pallas import tpu_sc as plsc

N = 10000
NP = 10240
E = 320000
EPB = 128
BPT = 80
NT = 32
EPT = BPT * EPB
EP = NT * EPT
SPT = NP // 16


def _sc_mesh():
    return plsc.VectorSubcoreMesh(core_axis_name="c", subcore_axis_name="s")


_SC_PARAMS = pltpu.CompilerParams(use_tc_tiling_on_sc=False)


@functools.partial(
    pl.kernel,
    out_type=jax.ShapeDtypeStruct((NT, SPT), jnp.float32),
    mesh=_sc_mesh(),
    compiler_params=_SC_PARAMS,
    scratch_types=[
        pltpu.VMEM((BPT, EPB), jnp.int32),
        pltpu.VMEM((EPB,), jnp.float32),
        pltpu.VMEM_SHARED((NP,), jnp.float32),
        pltpu.SemaphoreType.DMA,
    ],
)
def _deg_kernel(dst_hbm, ones_hbm, zeros_hbm, out_hbm, idx_d, ones_v, acc, sem):
    c = lax.axis_index("c")
    s = lax.axis_index("s")
    wid = c * 16 + s
    rsl = pl.ds(s * SPT, SPT)
    pltpu.sync_copy(zeros_hbm.at[rsl], acc.at[rsl])
    pltpu.sync_copy(ones_hbm, ones_v)
    pltpu.sync_copy(dst_hbm.at[pl.ds(wid * BPT, BPT)], idx_d)
    plsc.subcore_barrier()

    def body(j, carry):
        pltpu.async_copy(ones_v, acc.at[idx_d.at[j]], sem, add=True)
        return carry

    lax.fori_loop(0, BPT, body, 0)

    def drain(j, carry):
        pltpu.make_async_copy(ones_hbm, ones_v, sem).wait()
        return carry

    lax.fori_loop(0, BPT, drain, 0)
    plsc.subcore_barrier()
    pltpu.sync_copy(acc.at[rsl], out_hbm.at[wid])


def _make_agg(D):
    @functools.partial(
        pl.kernel,
        out_type=jax.ShapeDtypeStruct((NT, SPT, D), jnp.float32),
        mesh=_sc_mesh(),
        compiler_params=_SC_PARAMS,
        scratch_types=[
            pltpu.VMEM((BPT, EPB), jnp.int32),
            pltpu.VMEM((BPT, EPB), jnp.int32),
            [pltpu.VMEM((EPB, D), jnp.float32)] * 2,
            pltpu.VMEM_SHARED((NP, D), jnp.float32),
            pltpu.SemaphoreType.DMA,
            pltpu.SemaphoreType.DMA,
        ],
    )
    def agg(src_hbm, dst_hbm, y_hbm, zeros_hbm, out_hbm,
            idx_s, idx_d, rows, acc, sem, semi):
        c = lax.axis_index("c")
        s = lax.axis_index("s")
        wid = c * 16 + s
        rsl = pl.ds(s * SPT, SPT)
        pltpu.sync_copy(zeros_hbm.at[rsl], acc.at[rsl])
        h1 = pltpu.async_copy(src_hbm.at[pl.ds(wid * BPT, BPT)], idx_s, semi)
        h2 = pltpu.async_copy(dst_hbm.at[pl.ds(wid * BPT, BPT)], idx_d, semi)
        h1.wait()
        h2.wait()
        plsc.subcore_barrier()

        def drain2():
            pltpu.make_async_copy(y_hbm.at[pl.ds(0, EPB)], rows[0], semi).wait()
            pltpu.make_async_copy(y_hbm.at[pl.ds(0, EPB)], rows[1], semi).wait()

        def body(i, carry):
            j0 = i * 2

            @pl.when(i > 0)
            def _():
                drain2()

            g0 = pltpu.async_copy(y_hbm.at[idx_s.at[j0]], rows[0], sem)
            g1 = pltpu.async_copy(y_hbm.at[idx_s.at[j0 + 1]], rows[1], sem)
            g0.wait()
            pltpu.async_copy(rows[0], acc.at[idx_d.at[j0]], semi, add=True)
            g1.wait()
            pltpu.async_copy(rows[1], acc.at[idx_d.at[j0 + 1]], semi, add=True)
            return carry

        lax.fori_loop(0, BPT // 2, body, 0)
        drain2()
        plsc.subcore_barrier()
        pltpu.sync_copy(acc.at[rsl], out_hbm.at[wid])

    return agg


_agg64 = _make_agg(64)
_agg32 = _make_agg(32)


RB = 2000


def _dis(degp):
    return lax.rsqrt(degp[:, 0:1] + degp[:, 1:2] + 1.0)


def _mm_body(x_ref, w1_ref, xw_ref):
    xw_ref[...] = jnp.dot(
        x_ref[...], w1_ref[...], preferred_element_type=jnp.float32)


def _scale_body(degp_ref, xw_ref, y1_ref):
    y1_ref[...] = _dis(degp_ref[...]) * xw_ref[...]


def _mid_body(degp_ref, agg_ref, y1_ref, b1_ref, w2_ref, y2_ref):
    dis = _dis(degp_ref[...])
    a = agg_ref[0] + agg_ref[1] + y1_ref[...]
    h1 = jnp.maximum(dis * a + b1_ref[...], 0.0)
    y2_ref[...] = dis * jnp.dot(h1, w2_ref[...], preferred_element_type=jnp.float32)


def _post_body(degp_ref, agg_ref, y2_ref, b2_ref, wc_ref, bc_ref, out_ref):
    dis = _dis(degp_ref[...])
    a = agg_ref[0] + agg_ref[1] + y2_ref[...]
    h2 = jnp.maximum(dis * a + b2_ref[...], 0.0)
    out_ref[...] = (
        jnp.dot(h2, wc_ref[...], preferred_element_type=jnp.float32) + bc_ref[...])


def _row_spec(cols):
    return pl.BlockSpec((RB, cols), lambda i: (i, 0))


def _full_spec(shape):
    nd = len(shape)
    return pl.BlockSpec(shape, lambda i, _nd=nd: (0,) * _nd)


def _agg_spec(cols):
    return pl.BlockSpec((2, RB, cols), lambda i: (0, i, 0))


_GRID = N // RB

_mm = pl.pallas_call(
    _mm_body,
    grid=(_GRID,),
    in_specs=[_row_spec(128), _full_spec((128, 64))],
    out_specs=_row_spec(64),
    out_shape=jax.ShapeDtypeStruct((N, 64), jnp.float32),
)

_scale = pl.pallas_call(
    _scale_body,
    grid=(_GRID,),
    in_specs=[_row_spec(2), _row_spec(64)],
    out_specs=_row_spec(64),
    out_shape=jax.ShapeDtypeStruct((N, 64), jnp.float32),
)

_mid = pl.pallas_call(
    _mid_body,
    grid=(_GRID,),
    in_specs=[_row_spec(2), _agg_spec(64), _row_spec(64),
              _full_spec((1, 64)), _full_spec((64, 32))],
    out_specs=_row_spec(32),
    out_shape=jax.ShapeDtypeStruct((N, 32), jnp.float32),
)

_post = pl.pallas_call(
    _post_body,
    grid=(_GRID,),
    in_specs=[_row_spec(2), _agg_spec(32), _row_spec(32),
              _full_spec((1, 32)), _full_spec((32, 10)), _full_spec((1, 10))],
    out_specs=_row_spec(10),
    out_shape=jax.ShapeDtypeStruct((N, 10), jnp.float32),
)


def kernel(x, edge_index, W1, b1, W2, b2, Wc, bc):
    npad = EP - E
    pad_src = jnp.arange(npad, dtype=jnp.int32) % N
    pad_dst = N + jnp.arange(npad, dtype=jnp.int32) % (NP - N)
    src = jnp.concatenate([edge_index[0].astype(jnp.int32), pad_src])
    dst = jnp.concatenate([edge_index[1].astype(jnp.int32), pad_dst])
    src = src.reshape(NT * BPT, EPB)
    dst = dst.reshape(NT * BPT, EPB)

    ones = jnp.ones((EPB,), jnp.float32)
    zeros1 = jnp.zeros((NP,), jnp.float32)
    zeros64 = jnp.zeros((NP, 64), jnp.float32)
    zeros32 = jnp.zeros((NP, 32), jnp.float32)

    degp = _deg_kernel(dst, ones, zeros1)
    degp = degp.reshape(2, NP).T
    xw1 = _mm(x, W1)
    y1 = _scale(degp, xw1)
    agg1 = _agg64(src, dst, y1, zeros64).reshape(2, NP, 64)
    y2 = _mid(degp, agg1, y1, b1.reshape(1, 64), W2)
    agg2 = _agg32(src, dst, y2, zeros32).reshape(2, NP, 32)
    return _post(degp, agg2, y2, b2.reshape(1, 32), Wc, bc.reshape(1, 10))


# speedup vs baseline: 45.1537x; 1.1405x over previous
import functools

import jax
import jax.numpy as jnp
from jax import lax
from jax.experimental import pallas as pl
from jax.experimental.pallas import tpu as pltpu
from jax.experimental.pallas import tpu_sc as plsc

N = 10000
NP = 10240
E = 320000
EPB = 128
BPT = 80
NT = 32
EPT = BPT * EPB
EP = NT * EPT
SPT = NP // 16


def _sc_mesh():
    return plsc.VectorSubcoreMesh(core_axis_name="c", subcore_axis_name="s")


_SC_PARAMS = pltpu.CompilerParams(use_tc_tiling_on_sc=False)


@functools.partial(
    pl.kernel,
    out_type=jax.ShapeDtypeStruct((NT, SPT), jnp.float32),
    mesh=_sc_mesh(),
    compiler_params=_SC_PARAMS,
    scratch_types=[
        pltpu.VMEM((BPT, EPB), jnp.int32),
        pltpu.VMEM((EPB,), jnp.float32),
        pltpu.VMEM_SHARED((NP,), jnp.float32),
        pltpu.SemaphoreType.DMA,
    ],
)
def _deg_kernel(dst_hbm, ones_hbm, zeros_hbm, out_hbm, idx_d, ones_v, acc, sem):
    c = lax.axis_index("c")
    s = lax.axis_index("s")
    wid = c * 16 + s
    rsl = pl.ds(s * SPT, SPT)
    pltpu.sync_copy(zeros_hbm.at[rsl], acc.at[rsl])
    pltpu.sync_copy(ones_hbm, ones_v)
    pltpu.sync_copy(dst_hbm.at[pl.ds(wid * BPT, BPT)], idx_d)
    plsc.subcore_barrier()

    def body(j, carry):
        pltpu.async_copy(ones_v, acc.at[idx_d.at[j]], sem, add=True)
        return carry

    lax.fori_loop(0, BPT, body, 0)

    def drain(j, carry):
        pltpu.make_async_copy(ones_hbm, ones_v, sem).wait()
        return carry

    lax.fori_loop(0, BPT, drain, 0)
    plsc.subcore_barrier()
    pltpu.sync_copy(acc.at[rsl], out_hbm.at[wid])


def _make_agg(D):
    @functools.partial(
        pl.kernel,
        out_type=jax.ShapeDtypeStruct((NT, SPT, D), jnp.float32),
        mesh=_sc_mesh(),
        compiler_params=_SC_PARAMS,
        scratch_types=[
            pltpu.VMEM((BPT, EPB), jnp.int32),
            pltpu.VMEM((BPT, EPB), jnp.int32),
            [pltpu.VMEM((EPB, D), jnp.float32)] * 4,
            pltpu.VMEM_SHARED((NP, D), jnp.float32),
            pltpu.SemaphoreType.DMA,
            pltpu.SemaphoreType.DMA,
        ],
    )
    def agg(src_hbm, dst_hbm, y_hbm, zeros_hbm, out_hbm,
            idx_s, idx_d, rows, acc, sem, semi):
        c = lax.axis_index("c")
        s = lax.axis_index("s")
        wid = c * 16 + s
        rsl = pl.ds(s * SPT, SPT)
        pltpu.sync_copy(zeros_hbm.at[rsl], acc.at[rsl])
        h1 = pltpu.async_copy(src_hbm.at[pl.ds(wid * BPT, BPT)], idx_s, semi)
        h2 = pltpu.async_copy(dst_hbm.at[pl.ds(wid * BPT, BPT)], idx_d, semi)
        h1.wait()
        h2.wait()
        plsc.subcore_barrier()

        def drain4():
            for b in range(4):
                pltpu.make_async_copy(
                    y_hbm.at[pl.ds(0, EPB)], rows[b], semi).wait()

        def body(i, carry):
            j0 = i * 4

            @pl.when(i > 0)
            def _():
                drain4()

            gs = [pltpu.async_copy(y_hbm.at[idx_s.at[j0 + b]], rows[b], sem)
                  for b in range(4)]
            for b in range(4):
                gs[b].wait()
                pltpu.async_copy(rows[b], acc.at[idx_d.at[j0 + b]], semi,
                                 add=True)
            return carry

        lax.fori_loop(0, BPT // 4, body, 0)
        drain4()
        plsc.subcore_barrier()
        pltpu.sync_copy(acc.at[rsl], out_hbm.at[wid])

    return agg


_agg64 = _make_agg(64)
_agg32 = _make_agg(32)


RB = 2000


def _dis(degp):
    return lax.rsqrt(degp[:, 0:1] + degp[:, 1:2] + 1.0)


def _mm_body(x_ref, w1_ref, xw_ref):
    xw_ref[...] = jnp.dot(
        x_ref[...], w1_ref[...], preferred_element_type=jnp.float32)


def _scale_body(degp_ref, xw_ref, y1_ref):
    y1_ref[...] = _dis(degp_ref[...]) * xw_ref[...]


def _mid_body(degp_ref, agg_ref, y1_ref, b1_ref, w2_ref, y2_ref):
    dis = _dis(degp_ref[...])
    a = agg_ref[0] + agg_ref[1] + y1_ref[...]
    h1 = jnp.maximum(dis * a + b1_ref[...], 0.0)
    y2_ref[...] = dis * jnp.dot(h1, w2_ref[...], preferred_element_type=jnp.float32)


def _post_body(degp_ref, agg_ref, y2_ref, b2_ref, wc_ref, bc_ref, out_ref):
    dis = _dis(degp_ref[...])
    a = agg_ref[0] + agg_ref[1] + y2_ref[...]
    h2 = jnp.maximum(dis * a + b2_ref[...], 0.0)
    out_ref[...] = (
        jnp.dot(h2, wc_ref[...], preferred_element_type=jnp.float32) + bc_ref[...])


def _row_spec(cols):
    return pl.BlockSpec((RB, cols), lambda i: (i, 0))


def _full_spec(shape):
    nd = len(shape)
    return pl.BlockSpec(shape, lambda i, _nd=nd: (0,) * _nd)


def _agg_spec(cols):
    return pl.BlockSpec((2, RB, cols), lambda i: (0, i, 0))


_GRID = N // RB

_mm = pl.pallas_call(
    _mm_body,
    grid=(_GRID,),
    in_specs=[_row_spec(128), _full_spec((128, 64))],
    out_specs=_row_spec(64),
    out_shape=jax.ShapeDtypeStruct((N, 64), jnp.float32),
)

_scale = pl.pallas_call(
    _scale_body,
    grid=(_GRID,),
    in_specs=[_row_spec(2), _row_spec(64)],
    out_specs=_row_spec(64),
    out_shape=jax.ShapeDtypeStruct((N, 64), jnp.float32),
)

_mid = pl.pallas_call(
    _mid_body,
    grid=(_GRID,),
    in_specs=[_row_spec(2), _agg_spec(64), _row_spec(64),
              _full_spec((1, 64)), _full_spec((64, 32))],
    out_specs=_row_spec(32),
    out_shape=jax.ShapeDtypeStruct((N, 32), jnp.float32),
)

_post = pl.pallas_call(
    _post_body,
    grid=(_GRID,),
    in_specs=[_row_spec(2), _agg_spec(32), _row_spec(32),
              _full_spec((1, 32)), _full_spec((32, 10)), _full_spec((1, 10))],
    out_specs=_row_spec(10),
    out_shape=jax.ShapeDtypeStruct((N, 10), jnp.float32),
)


def kernel(x, edge_index, W1, b1, W2, b2, Wc, bc):
    npad = EP - E
    pad_src = jnp.arange(npad, dtype=jnp.int32) % N
    pad_dst = N + jnp.arange(npad, dtype=jnp.int32) % (NP - N)
    src = jnp.concatenate([edge_index[0].astype(jnp.int32), pad_src])
    dst = jnp.concatenate([edge_index[1].astype(jnp.int32), pad_dst])
    src = src.reshape(NT * BPT, EPB)
    dst = dst.reshape(NT * BPT, EPB)

    ones = jnp.ones((EPB,), jnp.float32)
    zeros1 = jnp.zeros((NP,), jnp.float32)
    zeros64 = jnp.zeros((NP, 64), jnp.float32)
    zeros32 = jnp.zeros((NP, 32), jnp.float32)

    degp = _deg_kernel(dst, ones, zeros1)
    degp = degp.reshape(2, NP).T
    xw1 = _mm(x, W1)
    y1 = _scale(degp, xw1)
    agg1 = _agg64(src, dst, y1, zeros64).reshape(2, NP, 64)
    y2 = _mid(degp, agg1, y1, b1.reshape(1, 64), W2)
    agg2 = _agg32(src, dst, y2, zeros32).reshape(2, NP, 32)
    return _post(degp, agg2, y2, b2.reshape(1, 32), Wc, bc.reshape(1, 10))


# speedup vs baseline: 46.5179x; 1.0302x over previous
import functools

import jax
import jax.numpy as jnp
from jax import lax
from jax.experimental import pallas as pl
from jax.experimental.pallas import tpu as pltpu
from jax.experimental.pallas import tpu_sc as plsc

N = 10000
NP = 10240
E = 320000
EPB = 256
BPT = 40
NT = 32
EPT = BPT * EPB
EP = NT * EPT
SPT = NP // 16


def _sc_mesh():
    return plsc.VectorSubcoreMesh(core_axis_name="c", subcore_axis_name="s")


_SC_PARAMS = pltpu.CompilerParams(use_tc_tiling_on_sc=False)


@functools.partial(
    pl.kernel,
    out_type=jax.ShapeDtypeStruct((NT, SPT), jnp.float32),
    mesh=_sc_mesh(),
    compiler_params=_SC_PARAMS,
    scratch_types=[
        pltpu.VMEM((BPT, EPB), jnp.int32),
        pltpu.VMEM((EPB,), jnp.float32),
        pltpu.VMEM_SHARED((NP,), jnp.float32),
        pltpu.SemaphoreType.DMA,
    ],
)
def _deg_kernel(dst_hbm, ones_hbm, zeros_hbm, out_hbm, idx_d, ones_v, acc, sem):
    c = lax.axis_index("c")
    s = lax.axis_index("s")
    wid = c * 16 + s
    rsl = pl.ds(s * SPT, SPT)
    pltpu.sync_copy(zeros_hbm.at[rsl], acc.at[rsl])
    pltpu.sync_copy(ones_hbm, ones_v)
    pltpu.sync_copy(dst_hbm.at[pl.ds(wid * BPT, BPT)], idx_d)
    plsc.subcore_barrier()

    def body(j, carry):
        pltpu.async_copy(ones_v, acc.at[idx_d.at[j]], sem, add=True)
        return carry

    lax.fori_loop(0, BPT, body, 0)

    def drain(j, carry):
        pltpu.make_async_copy(ones_hbm, ones_v, sem).wait()
        return carry

    lax.fori_loop(0, BPT, drain, 0)
    plsc.subcore_barrier()
    pltpu.sync_copy(acc.at[rsl], out_hbm.at[wid])


def _make_agg(D):
    @functools.partial(
        pl.kernel,
        out_type=jax.ShapeDtypeStruct((NT, SPT, D), jnp.float32),
        mesh=_sc_mesh(),
        compiler_params=_SC_PARAMS,
        scratch_types=[
            pltpu.VMEM((BPT, EPB), jnp.int32),
            pltpu.VMEM((BPT, EPB), jnp.int32),
            [pltpu.VMEM((EPB, D), jnp.float32)] * 4,
            pltpu.VMEM_SHARED((NP, D), jnp.float32),
            pltpu.SemaphoreType.DMA,
            pltpu.SemaphoreType.DMA,
        ],
    )
    def agg(src_hbm, dst_hbm, y_hbm, zeros_hbm, out_hbm,
            idx_s, idx_d, rows, acc, sem, semi):
        c = lax.axis_index("c")
        s = lax.axis_index("s")
        wid = c * 16 + s
        rsl = pl.ds(s * SPT, SPT)
        pltpu.sync_copy(zeros_hbm.at[rsl], acc.at[rsl])
        h1 = pltpu.async_copy(src_hbm.at[pl.ds(wid * BPT, BPT)], idx_s, semi)
        h2 = pltpu.async_copy(dst_hbm.at[pl.ds(wid * BPT, BPT)], idx_d, semi)
        h1.wait()
        h2.wait()
        plsc.subcore_barrier()

        def drain4():
            for b in range(4):
                pltpu.make_async_copy(
                    y_hbm.at[pl.ds(0, EPB)], rows[b], semi).wait()

        def body(i, carry):
            j0 = i * 4

            @pl.when(i > 0)
            def _():
                drain4()

            gs = [pltpu.async_copy(y_hbm.at[idx_s.at[j0 + b]], rows[b], sem)
                  for b in range(4)]
            for b in range(4):
                gs[b].wait()
                pltpu.async_copy(rows[b], acc.at[idx_d.at[j0 + b]], semi,
                                 add=True)
            return carry

        lax.fori_loop(0, BPT // 4, body, 0)
        drain4()
        plsc.subcore_barrier()
        pltpu.sync_copy(acc.at[rsl], out_hbm.at[wid])

    return agg


_agg64 = _make_agg(64)
_agg32 = _make_agg(32)


RB = 2000


def _dis(degp):
    return lax.rsqrt(degp[:, 0:1] + degp[:, 1:2] + 1.0)


def _mm_body(x_ref, w1_ref, xw_ref):
    xw_ref[...] = jnp.dot(
        x_ref[...], w1_ref[...], preferred_element_type=jnp.float32)


def _scale_body(degp_ref, xw_ref, y1_ref):
    y1_ref[...] = _dis(degp_ref[...]) * xw_ref[...]


def _mid_body(degp_ref, agg_ref, y1_ref, b1_ref, w2_ref, y2_ref):
    dis = _dis(degp_ref[...])
    a = agg_ref[0] + agg_ref[1] + y1_ref[...]
    h1 = jnp.maximum(dis * a + b1_ref[...], 0.0)
    y2_ref[...] = dis * jnp.dot(h1, w2_ref[...], preferred_element_type=jnp.float32)


def _post_body(degp_ref, agg_ref, y2_ref, b2_ref, wc_ref, bc_ref, out_ref):
    dis = _dis(degp_ref[...])
    a = agg_ref[0] + agg_ref[1] + y2_ref[...]
    h2 = jnp.maximum(dis * a + b2_ref[...], 0.0)
    out_ref[...] = (
        jnp.dot(h2, wc_ref[...], preferred_element_type=jnp.float32) + bc_ref[...])


def _row_spec(cols):
    return pl.BlockSpec((RB, cols), lambda i: (i, 0))


def _full_spec(shape):
    nd = len(shape)
    return pl.BlockSpec(shape, lambda i, _nd=nd: (0,) * _nd)


def _agg_spec(cols):
    return pl.BlockSpec((2, RB, cols), lambda i: (0, i, 0))


_GRID = N // RB

_mm = pl.pallas_call(
    _mm_body,
    grid=(_GRID,),
    in_specs=[_row_spec(128), _full_spec((128, 64))],
    out_specs=_row_spec(64),
    out_shape=jax.ShapeDtypeStruct((N, 64), jnp.float32),
)

_scale = pl.pallas_call(
    _scale_body,
    grid=(_GRID,),
    in_specs=[_row_spec(2), _row_spec(64)],
    out_specs=_row_spec(64),
    out_shape=jax.ShapeDtypeStruct((N, 64), jnp.float32),
)

_mid = pl.pallas_call(
    _mid_body,
    grid=(_GRID,),
    in_specs=[_row_spec(2), _agg_spec(64), _row_spec(64),
              _full_spec((1, 64)), _full_spec((64, 32))],
    out_specs=_row_spec(32),
    out_shape=jax.ShapeDtypeStruct((N, 32), jnp.float32),
)

_post = pl.pallas_call(
    _post_body,
    grid=(_GRID,),
    in_specs=[_row_spec(2), _agg_spec(32), _row_spec(32),
              _full_spec((1, 32)), _full_spec((32, 10)), _full_spec((1, 10))],
    out_specs=_row_spec(10),
    out_shape=jax.ShapeDtypeStruct((N, 10), jnp.float32),
)


def kernel(x, edge_index, W1, b1, W2, b2, Wc, bc):
    npad = EP - E
    pad_src = jnp.arange(npad, dtype=jnp.int32) % N
    pad_dst = N + jnp.arange(npad, dtype=jnp.int32) % (NP - N)
    src = jnp.concatenate([edge_index[0].astype(jnp.int32), pad_src])
    dst = jnp.concatenate([edge_index[1].astype(jnp.int32), pad_dst])
    src = src.reshape(NT * BPT, EPB)
    dst = dst.reshape(NT * BPT, EPB)

    ones = jnp.ones((EPB,), jnp.float32)
    zeros1 = jnp.zeros((NP,), jnp.float32)
    zeros64 = jnp.zeros((NP, 64), jnp.float32)
    zeros32 = jnp.zeros((NP, 32), jnp.float32)

    degp = _deg_kernel(dst, ones, zeros1)
    degp = degp.reshape(2, NP).T
    xw1 = _mm(x, W1)
    y1 = _scale(degp, xw1)
    agg1 = _agg64(src, dst, y1, zeros64).reshape(2, NP, 64)
    y2 = _mid(degp, agg1, y1, b1.reshape(1, 64), W2)
    agg2 = _agg32(src, dst, y2, zeros32).reshape(2, NP, 32)
    return _post(degp, agg2, y2, b2.reshape(1, 32), Wc, bc.reshape(1, 10))
